# Initial kernel scaffold; baseline (speedup 1.0000x reference)
import jax
import jax.numpy as jnp
from jax.experimental import pallas as pl


def kernel(xyz, feats, W1, b1, g1, be1, W2, b2, g2, be2):
    raise NotImplementedError("write your pallas kernel here")


# speedup vs baseline: 5.4369x; 5.4369x over previous
import functools

import jax
import jax.numpy as jnp
from jax import lax
from jax.experimental import pallas as pl
from jax.experimental.pallas import tpu as pltpu
from jax.experimental.pallas import tpu_sc as plsc

B, P, C_IN = 4, 8192, 16
M = P // 4
K = 32
C1, C2 = 32, 64
BM = 64
PB = P // (M // BM)
RB = 256
NW = 32
ROWS = B * M * K
CH = 128
EPS = 1e-5


def _k1_body(xyzt_ref, xyz_ref, featsT_ref, cen_ref, w1t_ref, b1_ref,
             idx_ref, cproj_ref, ptab_ref):
    xt = xyzt_ref[0]
    cen = cen_ref[0]
    pn = jnp.sum(xt * xt, axis=0, keepdims=True)
    cn = jnp.sum(cen * cen, axis=1, keepdims=True)
    cb = cen.astype(jnp.bfloat16).astype(jnp.float32)
    xb = xt.astype(jnp.bfloat16).astype(jnp.float32)
    dot = (cb[:, 0:1] * xb[0:1, :]
           + cb[:, 1:2] * xb[1:2, :]
           + cb[:, 2:3] * xb[2:3, :])
    d = cn + pn - 2.0 * dot
    iota = lax.broadcasted_iota(jnp.int32, (BM, P), 1)
    kiota = lax.broadcasted_iota(jnp.int32, (BM, K), 1)

    def step(k, carry):
        d, acc = carry
        m = jnp.min(d, axis=1, keepdims=True)
        cand = jnp.where(d == m, iota, P)
        ci = jnp.min(cand, axis=1, keepdims=True)
        acc = jnp.where(kiota == k, ci, acc)
        d = jnp.where(iota == ci, jnp.inf, d)
        return d, acc

    _, acc = lax.fori_loop(0, K, step, (d, jnp.zeros((BM, K), jnp.int32)))
    b = pl.program_id(0)
    idx_ref[0] = acc + b * P

    w1t = w1t_ref[...]
    cproj = (cen[:, 0:1] * w1t[0:1, :]
             + cen[:, 1:2] * w1t[1:2, :]
             + cen[:, 2:3] * w1t[2:3, :]) - b1_ref[...]
    cproj_ref[0] = cproj

    xb = xyz_ref[0]
    fb = featsT_ref[0]
    pt = (xb[:, 0:1] * w1t[0:1, :]
          + xb[:, 1:2] * w1t[1:2, :]
          + xb[:, 2:3] * w1t[2:3, :])
    pt = pt + jnp.dot(fb, w1t[3:, :], preferred_element_type=jnp.float32)
    ptab_ref[0] = pt


def _knn_project(xyz_t, xyz, featsT, centers, w1t, b1r):
    return pl.pallas_call(
        _k1_body,
        grid=(B, M // BM),
        in_specs=[
            pl.BlockSpec((1, 3, P), lambda b, i: (b, 0, 0)),
            pl.BlockSpec((1, PB, 3), lambda b, i: (b, i, 0)),
            pl.BlockSpec((1, PB, C_IN), lambda b, i: (b, i, 0)),
            pl.BlockSpec((1, BM, 3), lambda b, i: (b, i, 0)),
            pl.BlockSpec((3 + C_IN, C1), lambda b, i: (0, 0)),
            pl.BlockSpec((1, C1), lambda b, i: (0, 0)),
        ],
        out_specs=[
            pl.BlockSpec((1, BM, K), lambda b, i: (b, i, 0)),
            pl.BlockSpec((1, BM, C1), lambda b, i: (b, i, 0)),
            pl.BlockSpec((1, PB, C1), lambda b, i: (b, i, 0)),
        ],
        out_shape=[
            jax.ShapeDtypeStruct((B, M, K), jnp.int32),
            jax.ShapeDtypeStruct((B, M, C1), jnp.float32),
            jax.ShapeDtypeStruct((B, P, C1), jnp.float32),
        ],
    )(xyz_t, xyz, featsT, centers, w1t, b1r)


@functools.partial(
    pl.kernel,
    mesh=plsc.VectorSubcoreMesh(core_axis_name="c", subcore_axis_name="s"),
    compiler_params=pltpu.CompilerParams(use_tc_tiling_on_sc=False),
    out_type=jax.ShapeDtypeStruct((ROWS, C1), jnp.float32),
    scratch_types=[
        pltpu.VMEM((CH,), jnp.int32),
        pltpu.VMEM((CH, C1), jnp.float32),
        pltpu.SemaphoreType.DMA,
    ],
)
def _sc_gather(table_hbm, idx_hbm, out_hbm, idx_v, rows_v, sem):
    wid = lax.axis_index("s") * 2 + lax.axis_index("c")
    per_w = ROWS // NW
    base = wid * per_w

    def body(c, carry):
        off = base + c * CH
        pltpu.sync_copy(idx_hbm.at[pl.ds(off, CH)], idx_v)
        pltpu.async_copy(table_hbm.at[idx_v], rows_v, sem).wait()
        pltpu.sync_copy(rows_v, out_hbm.at[pl.ds(off, CH)])
        return carry

    lax.fori_loop(0, per_w // CH, body, 0)


def _k3_body(g_ref, cp_ref, sums_ref):
    h1 = g_ref[...] - cp_ref[...][:, None, :]
    s1 = jnp.sum(jnp.sum(h1, axis=0), axis=0)
    s2 = jnp.sum(jnp.sum(h1 * h1, axis=0), axis=0)

    @pl.when(pl.program_id(0) == 0)
    def _():
        sums_ref[...] = jnp.zeros_like(sums_ref)

    sums_ref[0:1, :] += s1[None, :]
    sums_ref[1:2, :] += s2[None, :]


def _k4_body(g_ref, cp_ref, a1_ref, s1_ref, w2t_ref, b2_ref, sums_ref):
    h1 = g_ref[...] - cp_ref[...][:, None, :]
    x1 = jnp.maximum(h1 * a1_ref[0][None, None, :]
                     + s1_ref[0][None, None, :], 0.0)
    x1f = x1.reshape(RB * K, C1)
    h2 = jnp.dot(x1f, w2t_ref[...], preferred_element_type=jnp.float32)
    h2 = h2 + b2_ref[...]
    s1 = jnp.sum(h2, axis=0)
    s2 = jnp.sum(h2 * h2, axis=0)

    @pl.when(pl.program_id(0) == 0)
    def _():
        sums_ref[...] = jnp.zeros_like(sums_ref)

    sums_ref[0:1, :] += s1[None, :]
    sums_ref[1:2, :] += s2[None, :]


def _k5_body(g_ref, cp_ref, a1_ref, s1_ref, w2t_ref, b2_ref, a2_ref, s2_ref,
             out_ref):
    h1 = g_ref[...] - cp_ref[...][:, None, :]
    x1 = jnp.maximum(h1 * a1_ref[0][None, None, :]
                     + s1_ref[0][None, None, :], 0.0)
    x1f = x1.reshape(RB * K, C1)
    h2 = jnp.dot(x1f, w2t_ref[...], preferred_element_type=jnp.float32)
    h2 = h2 + b2_ref[...]
    x2 = jnp.maximum(h2 * a2_ref[...] + s2_ref[...], 0.0)
    x3 = x2.reshape(RB, K, C2)
    mx = x3[:, 0, :]
    for k in range(1, K):
        mx = jnp.maximum(mx, x3[:, k, :])
    out_ref[...] = mx


def kernel(xyz, feats, W1, b1, g1, be1, W2, b2, g2, be2):
    idxc = jnp.linspace(0.0, P - 1, M).astype(jnp.int32)
    centers = jnp.take(xyz, idxc, axis=1)

    xyz_t = xyz.transpose(0, 2, 1)
    featsT = feats.transpose(0, 2, 1)
    w1t = W1.T
    b1r = b1.reshape(1, C1)

    idx, cproj, ptable = _knn_project(xyz_t, xyz, featsT, centers, w1t, b1r)

    g = _sc_gather(ptable.reshape(B * P, C1), idx.reshape(ROWS))
    g3 = g.reshape(B * M, K, C1)
    cpf = cproj.reshape(B * M, C1)

    nblk = (B * M) // RB
    sums1 = pl.pallas_call(
        _k3_body,
        grid=(nblk,),
        in_specs=[
            pl.BlockSpec((RB, K, C1), lambda i: (i, 0, 0)),
            pl.BlockSpec((RB, C1), lambda i: (i, 0)),
        ],
        out_specs=pl.BlockSpec((8, C1), lambda i: (0, 0)),
        out_shape=jax.ShapeDtypeStruct((8, C1), jnp.float32),
    )(g3, cpf)

    n1 = float(ROWS)
    mean1 = sums1[0] / n1
    var1 = sums1[1] / n1 - mean1 * mean1
    sc1 = g1 / jnp.sqrt(var1 + EPS)
    sh1 = be1 - mean1 * sc1
    w2t = W2.T
    b2r = b2.reshape(1, C2)

    sums2 = pl.pallas_call(
        _k4_body,
        grid=(nblk,),
        in_specs=[
            pl.BlockSpec((RB, K, C1), lambda i: (i, 0, 0)),
            pl.BlockSpec((RB, C1), lambda i: (i, 0)),
            pl.BlockSpec((1, C1), lambda i: (0, 0)),
            pl.BlockSpec((1, C1), lambda i: (0, 0)),
            pl.BlockSpec((C1, C2), lambda i: (0, 0)),
            pl.BlockSpec((1, C2), lambda i: (0, 0)),
        ],
        out_specs=pl.BlockSpec((8, C2), lambda i: (0, 0)),
        out_shape=jax.ShapeDtypeStruct((8, C2), jnp.float32),
    )(g3, cpf, sc1.reshape(1, C1), sh1.reshape(1, C1), w2t, b2r)

    mean2 = sums2[0] / n1
    var2 = sums2[1] / n1 - mean2 * mean2
    sc2 = g2 / jnp.sqrt(var2 + EPS)
    sh2 = be2 - mean2 * sc2

    out2 = pl.pallas_call(
        _k5_body,
        grid=(nblk,),
        in_specs=[
            pl.BlockSpec((RB, K, C1), lambda i: (i, 0, 0)),
            pl.BlockSpec((RB, C1), lambda i: (i, 0)),
            pl.BlockSpec((1, C1), lambda i: (0, 0)),
            pl.BlockSpec((1, C1), lambda i: (0, 0)),
            pl.BlockSpec((C1, C2), lambda i: (0, 0)),
            pl.BlockSpec((1, C2), lambda i: (0, 0)),
            pl.BlockSpec((1, C2), lambda i: (0, 0)),
            pl.BlockSpec((1, C2), lambda i: (0, 0)),
        ],
        out_specs=pl.BlockSpec((RB, C2), lambda i: (i, 0)),
        out_shape=jax.ShapeDtypeStruct((B * M, C2), jnp.float32),
    )(g3, cpf, sc1.reshape(1, C1), sh1.reshape(1, C1), w2t, b2r,
      sc2.reshape(1, C2), sh2.reshape(1, C2))

    out = out2.reshape(B, M, C2).transpose(0, 2, 1)
    return centers, out
